# Initial kernel scaffold; baseline (speedup 1.0000x reference)
import jax
import jax.numpy as jnp
from jax.experimental import pallas as pl


def kernel(rays_o, rays_d, rays_d_norm, near, far, table, beta_param):
    raise NotImplementedError("write your pallas kernel here")


# speedup vs baseline: 64.8511x; 64.8511x over previous
import functools

import jax
import jax.numpy as jnp
import numpy as np
from jax import lax
from jax.experimental import pallas as pl
from jax.experimental.pallas import tpu as pltpu
from jax.experimental.pallas import tpu_sc as plsc

_VOX = 0.05
_NUM_EMB = 40000
_S = 64
_R = 4096
_NC, _NS, _L = 2, 16, 16
_NW = _NC * _NS
_RAYS_W = _R // _NW
_CHUNKS = _RAYS_W // _L
_K1 = np.uint32(2654435761)
_K2 = np.uint32(805459861)


def _rsqrt(y):
    i = plsc.bitcast(y, jnp.int32)
    i = jnp.int32(0x5F3759DF) - (i >> 1)
    r = plsc.bitcast(i, jnp.float32)
    for _ in range(3):
        r = r * (1.5 - 0.5 * y * r * r)
    return r


def _tec_body(ch0_h, p12_h, ch3_h, rays_h, sc_h, grads_h, outs_h,
              ch0_v, p12_v, ch3_v, ray_v, sc_v, gst_v, ob_v, sem):
    wid = lax.axis_index("c") * _NS + lax.axis_index("s")
    rbase = pl.multiple_of(wid * _RAYS_W, _RAYS_W)

    cps = [
        pltpu.async_copy(ch0_h.at[:], ch0_v, sem),
        pltpu.async_copy(p12_h.at[:], p12_v, sem),
        pltpu.async_copy(ch3_h.at[:], ch3_v, sem),
        pltpu.async_copy(sc_h.at[:], sc_v, sem),
    ]
    for comp in range(6):
        cps.append(pltpu.async_copy(
            rays_h.at[pl.ds(comp * _R + rbase, _RAYS_W)],
            ray_v.at[pl.ds(comp * _RAYS_W, _RAYS_W)], sem))
    for c in cps:
        c.wait()

    near_v = sc_v[pl.ds(0, _L)]
    step_v = sc_v[pl.ds(_L, _L)]
    alpha_v = sc_v[pl.ds(2 * _L, _L)]
    iota192 = lax.iota(jnp.int32, _L) * (_S * 3)

    def chunk_body(ci, _):
        cb = pl.multiple_of(ci * _L, _L)
        o = [ray_v[pl.ds(comp * _RAYS_W + cb, _L)] for comp in range(3)]
        d = [ray_v[pl.ds((3 + comp) * _RAYS_W + cb, _L)]
             for comp in range(3)]

        def sbody(s, carry, o=o, d=d):
            T, cr0, cr1, cr2, cdep, cn0, cn1, cn2, cacc = carry
            sf = s.astype(jnp.float32)
            tn = near_v + sf * step_v
            tmid = 0.5 * (tn + (tn + step_v))
            fr = [None] * 3
            cu = [None] * 3
            for k in range(3):
                xc = (o[k] + tmid * d[k]) / jnp.float32(_VOX)
                ti = xc.astype(jnp.int32)
                tf = ti.astype(jnp.float32)
                neg = xc < tf
                negf = jnp.where(neg, jnp.float32(1.0), jnp.float32(0.0))
                negi = jnp.where(neg, jnp.int32(1), jnp.int32(0))
                fr[k] = xc - (tf - negf)
                cu[k] = plsc.bitcast(ti - negi, jnp.uint32)
            tx = [cu[0], cu[0] + jnp.uint32(1)]
            ty0 = cu[1] * _K1
            ty = [ty0, ty0 + _K1]
            tz0 = cu[2] * _K2
            tz = [tz0, tz0 + _K2]
            a = [1.0 - fr[0], fr[0]]
            b = [1.0 - fr[1], fr[1]]
            cc = [1.0 - fr[2], fr[2]]
            bc = [[b[j] * cc[k] for k in range(2)] for j in range(2)]
            ac = [[a[i] * cc[k] for k in range(2)] for i in range(2)]
            ab = [[a[i] * b[j] for j in range(2)] for i in range(2)]
            zero = jnp.zeros((_L,), jnp.float32)
            sdf, r0, r1, r2 = zero, zero, zero, zero
            e0s = {}
            for i in range(2):
                for j in range(2):
                    for k in range(2):
                        h = tx[i] ^ ty[j] ^ tz[k]
                        idx = (h % jnp.uint32(_NUM_EMB)).astype(jnp.int32)
                        e0 = plsc.load_gather(ch0_v, [idx])
                        pw = plsc.load_gather(p12_v, [idx])
                        e3 = plsc.load_gather(ch3_v, [idx])
                        e1 = plsc.bitcast(pw & jnp.int32(-65536), jnp.float32)
                        e2 = plsc.bitcast(pw << 16, jnp.float32)
                        w = a[i] * bc[j][k]
                        sdf = sdf + w * e0
                        r0 = r0 + w * e1
                        r1 = r1 + w * e2
                        r2 = r2 + w * e3
                        e0s[(i, j, k)] = e0
            gx = (bc[0][0] * (e0s[1, 0, 0] - e0s[0, 0, 0])
                  + bc[0][1] * (e0s[1, 0, 1] - e0s[0, 0, 1])
                  + bc[1][0] * (e0s[1, 1, 0] - e0s[0, 1, 0])
                  + bc[1][1] * (e0s[1, 1, 1] - e0s[0, 1, 1]))
            gy = (ac[0][0] * (e0s[0, 1, 0] - e0s[0, 0, 0])
                  + ac[0][1] * (e0s[0, 1, 1] - e0s[0, 0, 1])
                  + ac[1][0] * (e0s[1, 1, 0] - e0s[1, 0, 0])
                  + ac[1][1] * (e0s[1, 1, 1] - e0s[1, 0, 1]))
            gz = (ab[0][0] * (e0s[0, 0, 1] - e0s[0, 0, 0])
                  + ab[0][1] * (e0s[0, 1, 1] - e0s[0, 1, 0])
                  + ab[1][0] * (e0s[1, 0, 1] - e0s[1, 0, 0])
                  + ab[1][1] * (e0s[1, 1, 1] - e0s[1, 1, 0]))
            gx = gx / jnp.float32(_VOX)
            gy = gy / jnp.float32(_VOX)
            gz = gz / jnp.float32(_VOX)
            ib = iota192 + s * 3
            plsc.store_scatter(gst_v, [ib], gx)
            plsc.store_scatter(gst_v, [ib + 1], gy)
            plsc.store_scatter(gst_v, [ib + 2], gz)
            n2 = gx * gx + gy * gy + gz * gz
            rs = _rsqrt(jnp.maximum(n2, jnp.float32(1e-30)))
            inv = 1.0 / (n2 * rs + jnp.float32(1e-12))
            u = jnp.exp(-jnp.abs(sdf) * alpha_v)
            half_u = 0.5 * u
            sig = alpha_v * jnp.where(sdf >= 0, half_u, 1.0 - half_u)
            e = jnp.exp(-(sig * step_v))
            w = T * (1.0 - e)
            return (T * e,
                    cr0 + w * r0, cr1 + w * r1, cr2 + w * r2,
                    cdep + w * tmid,
                    cn0 + w * (gx * inv), cn1 + w * (gy * inv),
                    cn2 + w * (gz * inv),
                    cacc + w)

        init = ((jnp.ones((_L,), jnp.float32),)
                + tuple(jnp.zeros((_L,), jnp.float32) for _ in range(8)))
        res = lax.fori_loop(0, _S, sbody, init)
        for k in range(8):
            ob_v[pl.ds(k * _RAYS_W + cb, _L)] = res[k + 1]
        goff = pl.multiple_of((rbase + cb) * (_S * 3), _L * _S * 3)
        pltpu.sync_copy(gst_v, grads_h.at[pl.ds(goff, _L * _S * 3)])
        return 0

    lax.fori_loop(0, _CHUNKS, chunk_body, 0)

    ocps = [pltpu.async_copy(ob_v.at[pl.ds(k * _RAYS_W, _RAYS_W)],
                             outs_h.at[pl.ds(k * _R + rbase, _RAYS_W)], sem)
            for k in range(8)]
    for c in ocps:
        c.wait()


@functools.cache
def _sc_call():
    return pl.kernel(
        _tec_body,
        out_type=(jax.ShapeDtypeStruct((_R * _S * 3,), jnp.float32),
                  jax.ShapeDtypeStruct((8 * _R,), jnp.float32)),
        compiler_params=pltpu.CompilerParams(needs_layout_passes=False),
        mesh=plsc.VectorSubcoreMesh(core_axis_name="c",
                                    subcore_axis_name="s"),
        scratch_types=(
            pltpu.VMEM((_NUM_EMB,), jnp.float32),
            pltpu.VMEM((_NUM_EMB,), jnp.int32),
            pltpu.VMEM((_NUM_EMB,), jnp.float32),
            pltpu.VMEM((6 * _RAYS_W,), jnp.float32),
            pltpu.VMEM((3 * _L,), jnp.float32),
            pltpu.VMEM((_L * _S * 3,), jnp.float32),
            pltpu.VMEM((8 * _RAYS_W,), jnp.float32),
            pltpu.SemaphoreType.DMA,
        ),
    )


def kernel(rays_o, rays_d, rays_d_norm, near, far, table, beta_param):
    nearf = jnp.asarray(near).astype(jnp.float32)
    farf = jnp.asarray(far).astype(jnp.float32)
    step = (farf - nearf) / _S
    beta = jnp.float32(_VOX) + jnp.abs(beta_param[0])
    alpha = 1.0 / beta
    scv = jnp.concatenate([jnp.broadcast_to(nearf, (_L,)),
                           jnp.broadcast_to(step, (_L,)),
                           jnp.broadcast_to(alpha, (_L,))])

    ch0 = table[:, 0]
    ch3 = table[:, 3]
    b1 = lax.bitcast_convert_type(table[:, 1], jnp.uint32)
    b2 = lax.bitcast_convert_type(table[:, 2], jnp.uint32)
    p12 = lax.bitcast_convert_type(
        ((b1 + jnp.uint32(0x8000)) & jnp.uint32(0xFFFF0000))
        | ((b2 + jnp.uint32(0x8000)) >> 16), jnp.int32)

    raysf = jnp.concatenate([rays_o[:, 0], rays_o[:, 1], rays_o[:, 2],
                             rays_d[:, 0], rays_d[:, 1], rays_d[:, 2]])

    grads_flat, outs = _sc_call()(ch0, p12, ch3, raysf, scv)

    rgb = jnp.stack([outs[0:_R], outs[_R:2 * _R], outs[2 * _R:3 * _R]],
                    axis=1)
    depth = outs[3 * _R:4 * _R][:, None] / rays_d_norm
    normals = jnp.stack([outs[4 * _R:5 * _R], outs[5 * _R:6 * _R],
                         outs[6 * _R:7 * _R]], axis=1)
    acc = outs[7 * _R:8 * _R][:, None]
    sdf_grads = grads_flat.reshape(_R * _S, 3)
    near_out = jnp.broadcast_to(nearf, (_R, 1)) / rays_d_norm
    far_out = jnp.broadcast_to(farf, (_R, 1)) / rays_d_norm
    return (rgb, depth, normals, acc, sdf_grads, near_out, far_out)
